# Initial kernel scaffold; baseline (speedup 1.0000x reference)
import jax
import jax.numpy as jnp
from jax.experimental import pallas as pl


def kernel(query_id, kv_id, emb_weight):
    raise NotImplementedError("write your pallas kernel here")


# speedup vs baseline: 1.1126x; 1.1126x over previous
import jax
import jax.numpy as jnp
from jax.experimental import pallas as pl
from jax.experimental.pallas import tpu as pltpu

B, Q, KV, H = 1, 2048, 2048, 12
BQ = 512


def _bias_kernel(q_ref, kv_ref, w_ref, out_ref):
    h = pl.program_id(0)
    qv = q_ref[0, 0, :]
    kv = kv_ref[0, 0, :]
    ind = qv[:, None] == kv[None, :]
    w0 = w_ref[0, h]
    w1 = w_ref[1, h]
    out_ref[0, 0, :, :] = jnp.where(ind, w1, w0)


def kernel(query_id, kv_id, emb_weight):
    grid = (H, Q // BQ)
    out = pl.pallas_call(
        _bias_kernel,
        grid=grid,
        in_specs=[
            pl.BlockSpec((1, 1, BQ), lambda h, qi: (0, 0, qi)),
            pl.BlockSpec((1, 1, KV), lambda h, qi: (0, 0, 0)),
            pl.BlockSpec(memory_space=pltpu.SMEM),
        ],
        out_specs=pl.BlockSpec((1, 1, BQ, KV), lambda h, qi: (0, h, qi, 0)),
        out_shape=jax.ShapeDtypeStruct((B, H, Q, KV), jnp.float32),
    )(query_id, kv_id, emb_weight)
    return out
